# speedup vs baseline: 2.4747x; 2.4747x over previous
import functools

import jax
import jax.numpy as jnp
from jax import lax
from jax.experimental import pallas as pl
from jax.experimental.pallas import tpu as pltpu
from jax.experimental.pallas import tpu_sc as plsc

N_NODES = 10000
N_EDGES = 160000
IN_DIM = 256
HIDDEN = 256
OUT_DIM = 128

NC = 2
NS = 16
NW = NC * NS

N_PAD = 10240
E_PAD = 163840
EPW = E_PAD // NW
CHUNK = 128
NCHUNK = EPW // CHUNK
ROWS_PER_TILE = N_PAD // NS

_DENSE_BR = 1280


def _dense_body(f_ref, w0_ref, w1_ref, w2_ref, o_ref):
    h = jnp.dot(f_ref[...], w0_ref[...], preferred_element_type=jnp.float32)
    h = jnp.maximum(h, 0.0)
    h = jnp.dot(h, w1_ref[...], preferred_element_type=jnp.float32)
    h = jnp.maximum(h, 0.0)
    o_ref[...] = jnp.dot(h, w2_ref[...], preferred_element_type=jnp.float32)


def _dense(f, W0, W1, W2):
    grid = (N_PAD // _DENSE_BR,)
    return pl.pallas_call(
        _dense_body,
        grid=grid,
        in_specs=[
            pl.BlockSpec((_DENSE_BR, IN_DIM), lambda i: (i, 0)),
            pl.BlockSpec((IN_DIM, HIDDEN), lambda i: (0, 0)),
            pl.BlockSpec((HIDDEN, HIDDEN), lambda i: (0, 0)),
            pl.BlockSpec((HIDDEN, OUT_DIM), lambda i: (0, 0)),
        ],
        out_specs=pl.BlockSpec((_DENSE_BR, OUT_DIM), lambda i: (i, 0)),
        out_shape=jax.ShapeDtypeStruct((N_PAD, OUT_DIM), jnp.float32),
    )(f, W0, W1, W2)


def _combine_body(p_ref, o_ref):
    o_ref[...] = p_ref[0] + p_ref[1]


def _combine(p):
    grid = (N_PAD // _DENSE_BR,)
    return pl.pallas_call(
        _combine_body,
        grid=grid,
        in_specs=[pl.BlockSpec((NC, _DENSE_BR, OUT_DIM), lambda i: (0, i, 0))],
        out_specs=pl.BlockSpec((_DENSE_BR, OUT_DIM), lambda i: (i, 0)),
        out_shape=jax.ShapeDtypeStruct((N_PAD, OUT_DIM), jnp.float32),
    )(p)


_SC_MESH = plsc.VectorSubcoreMesh(core_axis_name="c", subcore_axis_name="s")


@functools.partial(
    pl.kernel,
    out_type=jax.ShapeDtypeStruct((NC, N_PAD, OUT_DIM), jnp.float32),
    mesh=_SC_MESH,
    scratch_types=[
        pltpu.VMEM((NCHUNK, CHUNK), jnp.int32),
        pltpu.VMEM((NCHUNK, CHUNK), jnp.int32),
        pltpu.VMEM((CHUNK, OUT_DIM), jnp.float32),
        pltpu.VMEM_SHARED((N_PAD, OUT_DIM), jnp.float32),
    ],
)
def _prop(h_hbm, src_hbm, dst_hbm, zeros_hbm, out_hbm, src_v, dst_v, rows_v, acc):
    c = lax.axis_index("c")
    s = lax.axis_index("s")
    wid = c * NS + s

    pltpu.sync_copy(zeros_hbm, acc.at[pl.ds(s * ROWS_PER_TILE, ROWS_PER_TILE)])
    pltpu.sync_copy(src_hbm.at[wid], src_v)
    pltpu.sync_copy(dst_hbm.at[wid], dst_v)
    plsc.subcore_barrier()

    @pl.loop(0, NCHUNK)
    def _(j):
        pltpu.sync_copy(h_hbm.at[src_v.at[j]], rows_v)
        pltpu.sync_copy(rows_v, acc.at[dst_v.at[j]], add=True)

    plsc.subcore_barrier()
    pltpu.sync_copy(acc.at[pl.ds(s * ROWS_PER_TILE, ROWS_PER_TILE)],
                    out_hbm.at[c, pl.ds(s * ROWS_PER_TILE, ROWS_PER_TILE)])


def kernel(features, edge_index, W0, W1, W2):
    f = jnp.pad(features, ((0, N_PAD - N_NODES), (0, 0)))
    pad_e = E_PAD - N_EDGES
    src = jnp.concatenate(
        [edge_index[0], jnp.full((pad_e,), N_NODES, jnp.int32)]
    ).reshape(NW, NCHUNK, CHUNK)
    dst = jnp.concatenate(
        [edge_index[1], jnp.full((pad_e,), N_NODES, jnp.int32)]
    ).reshape(NW, NCHUNK, CHUNK)
    zeros = jnp.zeros((ROWS_PER_TILE, OUT_DIM), jnp.float32)

    h = _dense(f, W0, W1, W2)
    for _ in range(3):
        partials = _prop(h, src, dst, zeros)
        h = _combine(partials)
    return h[:N_NODES]


# speedup vs baseline: 2.7188x; 1.0987x over previous
import functools

import jax
import jax.numpy as jnp
from jax import lax
from jax.experimental import pallas as pl
from jax.experimental.pallas import tpu as pltpu
from jax.experimental.pallas import tpu_sc as plsc

N_NODES = 10000
N_EDGES = 160000
IN_DIM = 256
HIDDEN = 256
OUT_DIM = 128

NC = 2
NS = 16
NW = NC * NS

N_PAD = 10240
E_PAD = 163840
EPW = E_PAD // NW
CHUNK = 128
NCHUNK = EPW // CHUNK
ROWS_PER_TILE = N_PAD // NS

_DENSE_BR = 1280


def _dense_body(f_ref, w0_ref, w1_ref, w2_ref, o_ref):
    h = jnp.dot(f_ref[...], w0_ref[...], preferred_element_type=jnp.float32)
    h = jnp.maximum(h, 0.0)
    h = jnp.dot(h, w1_ref[...], preferred_element_type=jnp.float32)
    h = jnp.maximum(h, 0.0)
    o_ref[...] = jnp.dot(h, w2_ref[...], preferred_element_type=jnp.float32)


def _dense(f, W0, W1, W2):
    grid = (N_PAD // _DENSE_BR,)
    return pl.pallas_call(
        _dense_body,
        grid=grid,
        in_specs=[
            pl.BlockSpec((_DENSE_BR, IN_DIM), lambda i: (i, 0)),
            pl.BlockSpec((IN_DIM, HIDDEN), lambda i: (0, 0)),
            pl.BlockSpec((HIDDEN, HIDDEN), lambda i: (0, 0)),
            pl.BlockSpec((HIDDEN, OUT_DIM), lambda i: (0, 0)),
        ],
        out_specs=pl.BlockSpec((_DENSE_BR, OUT_DIM), lambda i: (i, 0)),
        out_shape=jax.ShapeDtypeStruct((N_PAD, OUT_DIM), jnp.float32),
    )(f, W0, W1, W2)


def _combine_body(p_ref, o_ref):
    o_ref[...] = p_ref[0] + p_ref[1]


def _combine(p):
    grid = (N_PAD // _DENSE_BR,)
    return pl.pallas_call(
        _combine_body,
        grid=grid,
        in_specs=[pl.BlockSpec((NC, _DENSE_BR, OUT_DIM), lambda i: (0, i, 0))],
        out_specs=pl.BlockSpec((_DENSE_BR, OUT_DIM), lambda i: (i, 0)),
        out_shape=jax.ShapeDtypeStruct((N_PAD, OUT_DIM), jnp.float32),
    )(p)


_SC_MESH = plsc.VectorSubcoreMesh(core_axis_name="c", subcore_axis_name="s")


@functools.partial(
    pl.kernel,
    out_type=jax.ShapeDtypeStruct((NC, N_PAD, OUT_DIM), jnp.float32),
    mesh=_SC_MESH,
    scratch_types=[
        pltpu.VMEM((NCHUNK, CHUNK), jnp.int32),
        pltpu.VMEM((NCHUNK, CHUNK), jnp.int32),
        pltpu.VMEM((CHUNK, OUT_DIM), jnp.float32),
        pltpu.VMEM((CHUNK, OUT_DIM), jnp.float32),
        pltpu.VMEM_SHARED((N_PAD, OUT_DIM), jnp.float32),
        pltpu.SemaphoreType.DMA,
        pltpu.SemaphoreType.DMA,
    ],
)
def _prop(h_hbm, src_hbm, dst_hbm, zeros_hbm, out_hbm,
          src_v, dst_v, rows_a, rows_b, acc, sem_a, sem_b):
    c = lax.axis_index("c")
    s = lax.axis_index("s")
    wid = c * NS + s

    pltpu.sync_copy(zeros_hbm, acc.at[pl.ds(s * ROWS_PER_TILE, ROWS_PER_TILE)])
    pltpu.sync_copy(src_hbm.at[wid], src_v)
    pltpu.sync_copy(dst_hbm.at[wid], dst_v)
    plsc.subcore_barrier()

    pltpu.async_copy(h_hbm.at[src_v.at[0]], rows_a, sem_a)
    pltpu.async_copy(h_hbm.at[src_v.at[1]], rows_b, sem_b)

    @pl.loop(0, NCHUNK, step=2)
    def _(j):
        pltpu.make_async_copy(h_hbm.at[src_v.at[j]], rows_a, sem_a).wait()
        pltpu.sync_copy(rows_a, acc.at[dst_v.at[j]], add=True)

        @pl.when(j + 2 < NCHUNK)
        def _():
            pltpu.async_copy(h_hbm.at[src_v.at[j + 2]], rows_a, sem_a)

        pltpu.make_async_copy(h_hbm.at[src_v.at[j + 1]], rows_b, sem_b).wait()
        pltpu.sync_copy(rows_b, acc.at[dst_v.at[j + 1]], add=True)

        @pl.when(j + 3 < NCHUNK)
        def _():
            pltpu.async_copy(h_hbm.at[src_v.at[j + 3]], rows_b, sem_b)

    plsc.subcore_barrier()
    pltpu.sync_copy(acc.at[pl.ds(s * ROWS_PER_TILE, ROWS_PER_TILE)],
                    out_hbm.at[c, pl.ds(s * ROWS_PER_TILE, ROWS_PER_TILE)])


def kernel(features, edge_index, W0, W1, W2):
    f = jnp.pad(features, ((0, N_PAD - N_NODES), (0, 0)))
    pad_e = E_PAD - N_EDGES
    src = jnp.concatenate(
        [edge_index[0], jnp.full((pad_e,), N_NODES, jnp.int32)]
    ).reshape(NW, NCHUNK, CHUNK)
    dst = jnp.concatenate(
        [edge_index[1], jnp.full((pad_e,), N_NODES, jnp.int32)]
    ).reshape(NW, NCHUNK, CHUNK)
    zeros = jnp.zeros((ROWS_PER_TILE, OUT_DIM), jnp.float32)

    h = _dense(f, W0, W1, W2)
    for _ in range(3):
        partials = _prop(h, src, dst, zeros)
        h = _combine(partials)
    return h[:N_NODES]


# speedup vs baseline: 8.5478x; 3.1440x over previous
import functools

import jax
import jax.numpy as jnp
from jax import lax
from jax.experimental import pallas as pl
from jax.experimental.pallas import tpu as pltpu
from jax.experimental.pallas import tpu_sc as plsc

N_NODES = 10000
N_EDGES = 160000
IN_DIM = 256
HIDDEN = 256
OUT_DIM = 128

NC = 2
NS = 16
NW = NC * NS

N_PAD = 10240
E_PAD = 163840
EPW = E_PAD // NW
CHUNK = 128
NCHUNK = EPW // CHUNK
ROWS_PER_TILE = N_PAD // NS

_DENSE_BR = 1280


def _dense_body(f_ref, w0_ref, w1_ref, w2_ref, o_ref):
    h = jnp.dot(f_ref[...], w0_ref[...], preferred_element_type=jnp.float32)
    h = jnp.maximum(h, 0.0)
    h = jnp.dot(h, w1_ref[...], preferred_element_type=jnp.float32)
    h = jnp.maximum(h, 0.0)
    o_ref[...] = jnp.dot(h, w2_ref[...], preferred_element_type=jnp.float32)


def _dense(f, W0, W1, W2):
    grid = (N_PAD // _DENSE_BR,)
    return pl.pallas_call(
        _dense_body,
        grid=grid,
        in_specs=[
            pl.BlockSpec((_DENSE_BR, IN_DIM), lambda i: (i, 0)),
            pl.BlockSpec((IN_DIM, HIDDEN), lambda i: (0, 0)),
            pl.BlockSpec((HIDDEN, HIDDEN), lambda i: (0, 0)),
            pl.BlockSpec((HIDDEN, OUT_DIM), lambda i: (0, 0)),
        ],
        out_specs=pl.BlockSpec((_DENSE_BR, OUT_DIM), lambda i: (i, 0)),
        out_shape=jax.ShapeDtypeStruct((N_PAD, OUT_DIM), jnp.float32),
    )(f, W0, W1, W2)


def _combine_body(p_ref, o_ref):
    o_ref[...] = p_ref[0] + p_ref[1]


def _combine(p):
    grid = (N_PAD // _DENSE_BR,)
    return pl.pallas_call(
        _combine_body,
        grid=grid,
        in_specs=[pl.BlockSpec((NC, _DENSE_BR, OUT_DIM), lambda i: (0, i, 0))],
        out_specs=pl.BlockSpec((_DENSE_BR, OUT_DIM), lambda i: (i, 0)),
        out_shape=jax.ShapeDtypeStruct((N_PAD, OUT_DIM), jnp.float32),
    )(p)


_SC_MESH = plsc.VectorSubcoreMesh(core_axis_name="c", subcore_axis_name="s")


@functools.partial(
    pl.kernel,
    out_type=jax.ShapeDtypeStruct((NC, N_PAD, OUT_DIM), jnp.float32),
    mesh=_SC_MESH,
    scratch_types=[
        pltpu.VMEM((NCHUNK, CHUNK), jnp.int32),
        pltpu.VMEM((NCHUNK, CHUNK), jnp.int32),
        pltpu.VMEM((CHUNK, OUT_DIM), jnp.float32),
        pltpu.VMEM((CHUNK, OUT_DIM), jnp.float32),
        pltpu.VMEM_SHARED((N_PAD, OUT_DIM), jnp.float32),
        pltpu.SemaphoreType.DMA,
        pltpu.SemaphoreType.DMA,
    ],
)
def _prop(h_hbm, src_hbm, dst_hbm, zeros_hbm, out_hbm,
          src_v, dst_v, rows_a, rows_b, acc, sem_a, sem_b):
    c = lax.axis_index("c")
    s = lax.axis_index("s")
    wid = c * NS + s

    pltpu.sync_copy(zeros_hbm, acc.at[pl.ds(s * ROWS_PER_TILE, ROWS_PER_TILE)])
    pltpu.sync_copy(src_hbm.at[wid], src_v)
    pltpu.sync_copy(dst_hbm.at[wid], dst_v)
    plsc.subcore_barrier()

    pltpu.async_copy(h_hbm.at[src_v.at[0]], rows_a, sem_a)
    pltpu.async_copy(h_hbm.at[src_v.at[1]], rows_b, sem_b)

    @pl.loop(0, NCHUNK, step=2)
    def _(j):
        pltpu.make_async_copy(h_hbm.at[src_v.at[j]], rows_a, sem_a).wait()
        pltpu.sync_copy(rows_a, acc.at[dst_v.at[j]], add=True)

        @pl.when(j + 2 < NCHUNK)
        def _():
            pltpu.async_copy(h_hbm.at[src_v.at[j + 2]], rows_a, sem_a)

        pltpu.make_async_copy(h_hbm.at[src_v.at[j + 1]], rows_b, sem_b).wait()
        pltpu.sync_copy(rows_b, acc.at[dst_v.at[j + 1]], add=True)

        @pl.when(j + 3 < NCHUNK)
        def _():
            pltpu.async_copy(h_hbm.at[src_v.at[j + 3]], rows_b, sem_b)

    plsc.subcore_barrier()
    pltpu.sync_copy(acc.at[pl.ds(s * ROWS_PER_TILE, ROWS_PER_TILE)],
                    out_hbm.at[c, pl.ds(s * ROWS_PER_TILE, ROWS_PER_TILE)])


def kernel(features, edge_index, W0, W1, W2):
    f = jnp.pad(features, ((0, N_PAD - N_NODES), (0, 0)))
    pad_e = E_PAD - N_EDGES
    pad_idx = N_NODES + (jnp.arange(pad_e, dtype=jnp.int32) % (N_PAD - N_NODES))
    src = jnp.concatenate([edge_index[0], pad_idx]).reshape(NW, NCHUNK, CHUNK)
    dst = jnp.concatenate([edge_index[1], pad_idx]).reshape(NW, NCHUNK, CHUNK)
    zeros = jnp.zeros((ROWS_PER_TILE, OUT_DIM), jnp.float32)

    h = _dense(f, W0, W1, W2)
    for _ in range(3):
        partials = _prop(h, src, dst, zeros)
        h = _combine(partials)
    return h[:N_NODES]
